# baseline (speedup 1.0000x reference)
import functools

import jax
import jax.numpy as jnp
from jax import lax
from jax.experimental import pallas as pl
from jax.experimental.pallas import tpu as pltpu
from jax.experimental.pallas import tpu_sc as plsc

_K = 32768
_C = 4
_N = 65536
_TN = 1024
_TK = 1024
_NKCH = _K // _TK
_NTB = _N // _TN
_BIG = 2**30
_BETA = 0.25

_NFB = 8
_FB = _N // _NFB

_GR = 128
_NGR = _N // _GR


def _argmin_body(xf_ref, ek_ref, codes_ref, bd_ref, bi_ref):
    k = pl.program_id(1)
    xb = xf_ref[...]
    xsq = jnp.sum(xb * xb, axis=1)
    ech = ek_ref[0]
    esq = 0.25 * jnp.sum(ech * ech, axis=0)
    dot2 = lax.dot_general(
        xb, ech, (((1,), (0,)), ((), ())),
        preferred_element_type=jnp.float32,
        precision=lax.Precision.HIGHEST)
    d = (xsq[:, None] + esq[None, :]) - dot2
    bmin = jnp.min(d, axis=1)
    iota = lax.broadcasted_iota(jnp.int32, (_TN, _TK), 1)
    bidx = jnp.min(jnp.where(d == bmin[:, None], iota, 2**30),
                   axis=1) + k * _TK
    prev_d = jnp.where(k == 0, jnp.inf, bd_ref[...])
    prev_i = jnp.where(k == 0, 0, bi_ref[...])
    better = bmin < prev_d
    bd_ref[...] = jnp.where(better, bmin, prev_d)
    bi_ref[...] = jnp.where(better, bidx, prev_i)

    @pl.when(k == _NKCH - 1)
    def _():
        codes_ref[0, 0, :] = bi_ref[...]


def _codes_call(xf8, e2c):
    return pl.pallas_call(
        _argmin_body,
        grid=(_NTB, _NKCH),
        in_specs=[
            pl.BlockSpec((_TN, 8), lambda i, k: (i, 0)),
            pl.BlockSpec((1, 8, _TK), lambda i, k: (k, 0, 0)),
        ],
        out_specs=pl.BlockSpec((1, 1, _TN), lambda i, k: (i, 0, 0)),
        out_shape=jax.ShapeDtypeStruct((_NTB, 1, _TN), jnp.int32),
        scratch_shapes=[
            pltpu.VMEM((_TN,), jnp.float32),
            pltpu.VMEM((_TN,), jnp.int32),
        ],
    )(xf8, e2c)


def _gather_call(E2, codes_flat):
    info = plsc.get_sparse_core_info()
    nc, ns = info.num_cores, info.num_subcores
    nw = nc * ns
    tpw = _N // nw
    half = _K // 2
    mesh = plsc.VectorSubcoreMesh(core_axis_name="c", subcore_axis_name="s")

    @functools.partial(
        pl.kernel, mesh=mesh,
        compiler_params=pltpu.CompilerParams(use_tc_tiling_on_sc=False, needs_layout_passes=False),
        out_type=jax.ShapeDtypeStruct((nw * _C, tpw), jnp.float32),
        scratch_types=[
            pltpu.VMEM((tpw,), jnp.int32),
            pltpu.VMEM((half * _C // 128, 128), jnp.float32),
            pltpu.VMEM((_C, tpw), jnp.float32),
        ],
    )
    def gather_k(e2_hbm, codes_hbm, out_hbm, codes_v, et, outb):
        wid = lax.axis_index("s") * nc + lax.axis_index("c")
        pltpu.sync_copy(codes_hbm.at[pl.ds(wid * tpw, tpw)], codes_v)
        for p in range(2):
            pltpu.sync_copy(e2_hbm.at[pl.ds(p * (half * _C // 128),
                                            half * _C // 128)], et)

            def body(i, _):
                code = codes_v[pl.ds(i * 16, 16)]
                local = code - p * half
                lc = jnp.minimum(jnp.maximum(local, 0), half - 1)
                valid = (local >= 0) & (local < half)
                row = lc >> 5
                colbase = (lc & 31) * 4
                for c in range(4):
                    g = plsc.load_gather(et, [row, colbase + c])
                    if p == 0:
                        val = jnp.where(valid, g, 0.0)
                    else:
                        val = jnp.where(valid, g, outb[c, pl.ds(i * 16, 16)])
                    outb[c, pl.ds(i * 16, 16)] = val
                return 0

            lax.fori_loop(0, tpw // 16, body, 0)
        pltpu.sync_copy(outb, out_hbm.at[pl.ds(wid * _C, _C)])

    out = gather_k(E2, codes_flat)
    return out.reshape(nw, _C, tpw).transpose(0, 2, 1).reshape(_N, _C)


def _finish_body(xf_ref, zq_ref, zst_ref, q_ref, acc_ref):
    i = pl.program_id(0)
    xb = xf_ref[...]
    diff = zq_ref[...] - xb
    zst_ref[...] = xb + diff
    part = jnp.sum(diff * diff)

    @pl.when(i == 0)
    def _():
        acc_ref[0, 0] = part

    @pl.when(i > 0)
    def _():
        acc_ref[0, 0] += part

    @pl.when(i == _NFB - 1)
    def _():
        m = acc_ref[0, 0] * (1.0 / float(_N * _C))
        q_ref[0, 0] = m + _BETA * m


def _finish_call(xf, zq):
    return pl.pallas_call(
        _finish_body,
        grid=(_NFB,),
        in_specs=[
            pl.BlockSpec((_FB, _C), lambda i: (i, 0)),
            pl.BlockSpec((_FB, _C), lambda i: (i, 0)),
        ],
        out_specs=[
            pl.BlockSpec((_FB, _C), lambda i: (i, 0)),
            pl.BlockSpec(memory_space=pltpu.SMEM),
        ],
        out_shape=[
            jax.ShapeDtypeStruct((_N, _C), jnp.float32),
            jax.ShapeDtypeStruct((1, 1), jnp.float32),
        ],
        scratch_shapes=[pltpu.SMEM((1, 1), jnp.float32)],
    )(xf, zq)


def kernel(x, E):
    b, t, c, h, w = x.shape
    xf = jnp.transpose(x, (0, 1, 3, 4, 2)).reshape(-1, c)
    xf8 = jnp.concatenate([xf, jnp.zeros((_N, 8 - _C), jnp.float32)], axis=1)
    e2t = (2.0 * E).T.reshape(_C, _NKCH, _TK).transpose(1, 0, 2)
    e2c = jnp.concatenate(
        [e2t, jnp.zeros((_NKCH, 8 - _C, _TK), jnp.float32)], axis=1)

    codes3 = _codes_call(xf8, e2c)
    codes_flat = codes3.reshape(_N)

    zq_flat = _gather_call(E.reshape(_K * _C // 128, 128), codes_flat)

    zst_flat, q = _finish_call(xf, zq_flat)

    z_q_st = zst_flat.reshape(b, t, h, w, c).transpose(0, 1, 4, 2, 3)
    qloss = jnp.reshape(q, ())
    codes = codes_flat.reshape(b, t, h, w)
    return z_q_st, qloss, codes


# speedup vs baseline: 1.6306x; 1.6306x over previous
import functools

import jax
import jax.numpy as jnp
from jax import lax
from jax.experimental import pallas as pl
from jax.experimental.pallas import tpu as pltpu
from jax.experimental.pallas import tpu_sc as plsc

_K = 32768
_C = 4
_N = 65536
_TN = 1024
_TK = 1024
_NKCH = _K // _TK
_NTB = _N // _TN
_BIG = 2**30
_BETA = 0.25

_NFB = 8
_FB = _N // _NFB

_GR = 128
_NGR = _N // _GR


def _argmin_body(xt_ref, ek_ref, codes_ref, bd_ref, bi_ref):
    k = pl.program_id(1)
    xb = xt_ref[...]
    xsq = jnp.sum(xb * xb, axis=0)
    ech = ek_ref[0]
    esqc = 0.25 * jnp.sum(ech * ech, axis=1,
                          keepdims=True)
    dot2 = lax.dot_general(
        ech, xb, (((1,), (0,)), ((), ())),
        preferred_element_type=jnp.float32,
        precision=lax.Precision.HIGHEST)
    d = (xsq[None, :] + esqc) - dot2
    dr = d.reshape(_TK // 8, 8, _TN)
    bminc = jnp.min(dr, axis=0)
    viota = lax.broadcasted_iota(jnp.int32, (_TK // 8, 8, _TN), 0)
    vmin = jnp.min(jnp.where(dr == bminc[None], viota, 2**30),
                   axis=0)
    siota = lax.broadcasted_iota(jnp.int32, (8, _TN), 0)
    bidxc = k * _TK + (vmin * 8 + siota)
    prev_d = jnp.where(k == 0, jnp.inf, bd_ref[...])
    prev_i = jnp.where(k == 0, 0, bi_ref[...])
    better = bminc < prev_d
    bd_ref[...] = jnp.where(better, bminc, prev_d)
    bi_ref[...] = jnp.where(better, bidxc, prev_i)

    @pl.when(k == _NKCH - 1)
    def _():
        av = bd_ref[...]
        ai = bi_ref[...]
        bfin = jnp.min(av, axis=0)
        codes_ref[0, 0, :] = jnp.min(
            jnp.where(av == bfin[None, :], ai, 2**30), axis=0)


def _codes_call(xt8, e2r):
    return pl.pallas_call(
        _argmin_body,
        grid=(_NTB, _NKCH),
        in_specs=[
            pl.BlockSpec((8, _TN), lambda i, k: (0, i)),
            pl.BlockSpec((1, _TK, 8), lambda i, k: (k, 0, 0)),
        ],
        out_specs=pl.BlockSpec((1, 1, _TN), lambda i, k: (i, 0, 0)),
        out_shape=jax.ShapeDtypeStruct((_NTB, 1, _TN), jnp.int32),
        scratch_shapes=[
            pltpu.VMEM((8, _TN), jnp.float32),
            pltpu.VMEM((8, _TN), jnp.int32),
        ],
    )(xt8, e2r)


def _gather_call(E2, codes_flat):
    info = plsc.get_sparse_core_info()
    nc, ns = info.num_cores, info.num_subcores
    nw = nc * ns
    tpw = _N // nw
    half = _K // 2
    mesh = plsc.VectorSubcoreMesh(core_axis_name="c", subcore_axis_name="s")

    @functools.partial(
        pl.kernel, mesh=mesh,
        compiler_params=pltpu.CompilerParams(use_tc_tiling_on_sc=False, needs_layout_passes=False),
        out_type=jax.ShapeDtypeStruct((nw * _C, tpw), jnp.float32),
        scratch_types=[
            pltpu.VMEM((tpw,), jnp.int32),
            pltpu.VMEM((half * _C // 128, 128), jnp.float32),
            pltpu.VMEM((_C, tpw), jnp.float32),
        ],
    )
    def gather_k(e2_hbm, codes_hbm, out_hbm, codes_v, et, outb):
        wid = lax.axis_index("s") * nc + lax.axis_index("c")
        pltpu.sync_copy(codes_hbm.at[pl.ds(wid * tpw, tpw)], codes_v)
        for p in range(2):
            pltpu.sync_copy(e2_hbm.at[pl.ds(p * (half * _C // 128),
                                            half * _C // 128)], et)

            def body(i, _):
                code = codes_v[pl.ds(i * 16, 16)]
                local = code - p * half
                lc = jnp.minimum(jnp.maximum(local, 0), half - 1)
                valid = (local >= 0) & (local < half)
                row = lc >> 5
                colbase = (lc & 31) * 4
                for c in range(4):
                    g = plsc.load_gather(et, [row, colbase + c])
                    if p == 0:
                        val = jnp.where(valid, g, 0.0)
                    else:
                        val = jnp.where(valid, g, outb[c, pl.ds(i * 16, 16)])
                    outb[c, pl.ds(i * 16, 16)] = val
                return 0

            lax.fori_loop(0, tpw // 16, body, 0)
        pltpu.sync_copy(outb, out_hbm.at[pl.ds(wid * _C, _C)])

    out = gather_k(E2, codes_flat)
    return out.reshape(nw, _C, tpw).transpose(0, 2, 1).reshape(_N, _C)


def _finish_body(xf_ref, zq_ref, zst_ref, q_ref, acc_ref):
    i = pl.program_id(0)
    xb = xf_ref[...]
    diff = zq_ref[...] - xb
    zst_ref[...] = xb + diff
    part = jnp.sum(diff * diff)

    @pl.when(i == 0)
    def _():
        acc_ref[0, 0] = part

    @pl.when(i > 0)
    def _():
        acc_ref[0, 0] += part

    @pl.when(i == _NFB - 1)
    def _():
        m = acc_ref[0, 0] * (1.0 / float(_N * _C))
        q_ref[0, 0] = m + _BETA * m


def _finish_call(xf, zq):
    return pl.pallas_call(
        _finish_body,
        grid=(_NFB,),
        in_specs=[
            pl.BlockSpec((_FB, _C), lambda i: (i, 0)),
            pl.BlockSpec((_FB, _C), lambda i: (i, 0)),
        ],
        out_specs=[
            pl.BlockSpec((_FB, _C), lambda i: (i, 0)),
            pl.BlockSpec(memory_space=pltpu.SMEM),
        ],
        out_shape=[
            jax.ShapeDtypeStruct((_N, _C), jnp.float32),
            jax.ShapeDtypeStruct((1, 1), jnp.float32),
        ],
        scratch_shapes=[pltpu.SMEM((1, 1), jnp.float32)],
    )(xf, zq)


def kernel(x, E):
    b, t, c, h, w = x.shape
    xf = jnp.transpose(x, (0, 1, 3, 4, 2)).reshape(-1, c)
    xt8 = jnp.concatenate(
        [xf.T, jnp.zeros((8 - _C, _N), jnp.float32)], axis=0)
    e2r = jnp.concatenate(
        [(2.0 * E).reshape(_NKCH, _TK, _C),
         jnp.zeros((_NKCH, _TK, 8 - _C), jnp.float32)], axis=2)

    codes3 = _codes_call(xt8, e2r)
    codes_flat = codes3.reshape(_N)

    zq_flat = _gather_call(E.reshape(_K * _C // 128, 128), codes_flat)

    zst_flat, q = _finish_call(xf, zq_flat)

    z_q_st = zst_flat.reshape(b, t, h, w, c).transpose(0, 1, 4, 2, 3)
    qloss = jnp.reshape(q, ())
    codes = codes_flat.reshape(b, t, h, w)
    return z_q_st, qloss, codes


# speedup vs baseline: 1.8457x; 1.1319x over previous
import functools

import jax
import jax.numpy as jnp
from jax import lax
from jax.experimental import pallas as pl
from jax.experimental.pallas import tpu as pltpu
from jax.experimental.pallas import tpu_sc as plsc

_K = 32768
_C = 4
_N = 65536
_TN = 1024
_TK = 1024
_NKCH = _K // _TK
_NTB = _N // _TN
_BIG = 2**30
_BETA = 0.25

_NFB = 8
_FB = _N // _NFB

_GR = 128
_NGR = _N // _GR


def _argmin_body(xt_ref, ek_ref, codes_ref, bd_ref, bi_ref):
    k = pl.program_id(1)
    xb = xt_ref[...]
    xsq = jnp.sum(xb * xb, axis=0)
    ech = ek_ref[0]
    esqc = 0.25 * jnp.sum(ech * ech, axis=1,
                          keepdims=True)
    dot2 = lax.dot_general(
        ech, xb, (((1,), (0,)), ((), ())),
        preferred_element_type=jnp.float32,
        precision=lax.Precision.HIGHEST)
    accv = jnp.full((8, _TN), jnp.inf, jnp.float32)
    acci = jnp.zeros((8, _TN), jnp.int32)
    for v in range(_TK // 8):
        dv = (xsq[None, :] + esqc[v * 8:(v + 1) * 8, :]) \
            - dot2[v * 8:(v + 1) * 8, :]
        mask = dv < accv
        accv = jnp.where(mask, dv, accv)
        acci = jnp.where(mask, v, acci)
    bminc = accv
    siota = lax.broadcasted_iota(jnp.int32, (8, _TN), 0)
    bidxc = k * _TK + (acci * 8 + siota)
    prev_d = jnp.where(k == 0, jnp.inf, bd_ref[...])
    prev_i = jnp.where(k == 0, 0, bi_ref[...])
    better = bminc < prev_d
    bd_ref[...] = jnp.where(better, bminc, prev_d)
    bi_ref[...] = jnp.where(better, bidxc, prev_i)

    @pl.when(k == _NKCH - 1)
    def _():
        av = bd_ref[...]
        ai = bi_ref[...]
        bfin = jnp.min(av, axis=0)
        codes_ref[0, 0, :] = jnp.min(
            jnp.where(av == bfin[None, :], ai, 2**30), axis=0)


def _codes_call(xt8, e2r):
    return pl.pallas_call(
        _argmin_body,
        grid=(_NTB, _NKCH),
        in_specs=[
            pl.BlockSpec((8, _TN), lambda i, k: (0, i)),
            pl.BlockSpec((1, _TK, 8), lambda i, k: (k, 0, 0)),
        ],
        out_specs=pl.BlockSpec((1, 1, _TN), lambda i, k: (i, 0, 0)),
        out_shape=jax.ShapeDtypeStruct((_NTB, 1, _TN), jnp.int32),
        scratch_shapes=[
            pltpu.VMEM((8, _TN), jnp.float32),
            pltpu.VMEM((8, _TN), jnp.int32),
        ],
    )(xt8, e2r)


def _gather_call(E2, codes_flat):
    info = plsc.get_sparse_core_info()
    nc, ns = info.num_cores, info.num_subcores
    nw = nc * ns
    tpw = _N // nw
    half = _K // 2
    mesh = plsc.VectorSubcoreMesh(core_axis_name="c", subcore_axis_name="s")

    @functools.partial(
        pl.kernel, mesh=mesh,
        compiler_params=pltpu.CompilerParams(use_tc_tiling_on_sc=False, needs_layout_passes=False),
        out_type=jax.ShapeDtypeStruct((nw * _C, tpw), jnp.float32),
        scratch_types=[
            pltpu.VMEM((tpw,), jnp.int32),
            pltpu.VMEM((half * _C // 128, 128), jnp.float32),
            pltpu.VMEM((_C, tpw), jnp.float32),
        ],
    )
    def gather_k(e2_hbm, codes_hbm, out_hbm, codes_v, et, outb):
        wid = lax.axis_index("s") * nc + lax.axis_index("c")
        pltpu.sync_copy(codes_hbm.at[pl.ds(wid * tpw, tpw)], codes_v)
        for p in range(2):
            pltpu.sync_copy(e2_hbm.at[pl.ds(p * (half * _C // 128),
                                            half * _C // 128)], et)

            def body(i, _):
                code = codes_v[pl.ds(i * 16, 16)]
                local = code - p * half
                lc = jnp.minimum(jnp.maximum(local, 0), half - 1)
                valid = (local >= 0) & (local < half)
                row = lc >> 5
                colbase = (lc & 31) * 4
                for c in range(4):
                    g = plsc.load_gather(et, [row, colbase + c])
                    if p == 0:
                        val = jnp.where(valid, g, 0.0)
                    else:
                        val = jnp.where(valid, g, outb[c, pl.ds(i * 16, 16)])
                    outb[c, pl.ds(i * 16, 16)] = val
                return 0

            lax.fori_loop(0, tpw // 16, body, 0)
        pltpu.sync_copy(outb, out_hbm.at[pl.ds(wid * _C, _C)])

    out = gather_k(E2, codes_flat)
    return out.reshape(nw, _C, tpw).transpose(0, 2, 1).reshape(_N, _C)


def _finish_body(xf_ref, zq_ref, zst_ref, q_ref, acc_ref):
    i = pl.program_id(0)
    xb = xf_ref[...]
    diff = zq_ref[...] - xb
    zst_ref[...] = xb + diff
    part = jnp.sum(diff * diff)

    @pl.when(i == 0)
    def _():
        acc_ref[0, 0] = part

    @pl.when(i > 0)
    def _():
        acc_ref[0, 0] += part

    @pl.when(i == _NFB - 1)
    def _():
        m = acc_ref[0, 0] * (1.0 / float(_N * _C))
        q_ref[0, 0] = m + _BETA * m


def _finish_call(xf, zq):
    return pl.pallas_call(
        _finish_body,
        grid=(_NFB,),
        in_specs=[
            pl.BlockSpec((_FB, _C), lambda i: (i, 0)),
            pl.BlockSpec((_FB, _C), lambda i: (i, 0)),
        ],
        out_specs=[
            pl.BlockSpec((_FB, _C), lambda i: (i, 0)),
            pl.BlockSpec(memory_space=pltpu.SMEM),
        ],
        out_shape=[
            jax.ShapeDtypeStruct((_N, _C), jnp.float32),
            jax.ShapeDtypeStruct((1, 1), jnp.float32),
        ],
        scratch_shapes=[pltpu.SMEM((1, 1), jnp.float32)],
    )(xf, zq)


def kernel(x, E):
    b, t, c, h, w = x.shape
    xf = jnp.transpose(x, (0, 1, 3, 4, 2)).reshape(-1, c)
    xt8 = jnp.concatenate(
        [xf.T, jnp.zeros((8 - _C, _N), jnp.float32)], axis=0)
    e2r = jnp.concatenate(
        [(2.0 * E).reshape(_NKCH, _TK, _C),
         jnp.zeros((_NKCH, _TK, 8 - _C), jnp.float32)], axis=2)

    codes3 = _codes_call(xt8, e2r)
    codes_flat = codes3.reshape(_N)

    zq_flat = _gather_call(E.reshape(_K * _C // 128, 128), codes_flat)

    zst_flat, q = _finish_call(xf, zq_flat)

    z_q_st = zst_flat.reshape(b, t, h, w, c).transpose(0, 1, 4, 2, 3)
    qloss = jnp.reshape(q, ())
    codes = codes_flat.reshape(b, t, h, w)
    return z_q_st, qloss, codes


# speedup vs baseline: 6.0857x; 3.2973x over previous
import functools

import jax
import jax.numpy as jnp
from jax import lax
from jax.experimental import pallas as pl
from jax.experimental.pallas import tpu as pltpu
from jax.experimental.pallas import tpu_sc as plsc

_K = 32768
_C = 4
_N = 65536
_TN = 1024
_TK = 1024
_NKCH = _K // _TK
_NTB = _N // _TN
_BIG = 2**30
_BETA = 0.25

_NFB = 8
_FB = _N // _NFB

_GR = 128
_NGR = _N // _GR


def _argmin_body(xt_ref, ek_ref, codes_ref, bd_ref, bi_ref):
    k = pl.program_id(1)
    xb = xt_ref[...]
    xsq = jnp.sum(xb * xb, axis=0)
    ech = ek_ref[0]
    esqc = 0.25 * jnp.sum(ech * ech, axis=1,
                          keepdims=True)
    xb16 = xb.astype(jnp.bfloat16)
    dot2 = lax.dot_general(
        ech, xb16, (((1,), (0,)), ((), ())),
        preferred_element_type=jnp.float32)
    accv = jnp.full((8, _TN), jnp.inf, jnp.float32)
    acci = jnp.zeros((8, _TN), jnp.int32)
    for v in range(_TK // 8):
        dv = (xsq[None, :] + esqc[v * 8:(v + 1) * 8, :]) \
            - dot2[v * 8:(v + 1) * 8, :]
        mask = dv < accv
        accv = jnp.where(mask, dv, accv)
        acci = jnp.where(mask, v, acci)
    bminc = accv
    siota = lax.broadcasted_iota(jnp.int32, (8, _TN), 0)
    bidxc = k * _TK + (acci * 8 + siota)
    prev_d = jnp.where(k == 0, jnp.inf, bd_ref[...])
    prev_i = jnp.where(k == 0, 0, bi_ref[...])
    better = bminc < prev_d
    bd_ref[...] = jnp.where(better, bminc, prev_d)
    bi_ref[...] = jnp.where(better, bidxc, prev_i)

    @pl.when(k == _NKCH - 1)
    def _():
        av = bd_ref[...]
        ai = bi_ref[...]
        bfin = jnp.min(av, axis=0)
        codes_ref[0, 0, :] = jnp.min(
            jnp.where(av == bfin[None, :], ai, 2**30), axis=0)


def _codes_call(xt8, e2r):
    return pl.pallas_call(
        _argmin_body,
        grid=(_NTB, _NKCH),
        in_specs=[
            pl.BlockSpec((8, _TN), lambda i, k: (0, i)),
            pl.BlockSpec((1, _TK, 8), lambda i, k: (k, 0, 0)),
        ],
        out_specs=pl.BlockSpec((1, 1, _TN), lambda i, k: (i, 0, 0)),
        out_shape=jax.ShapeDtypeStruct((_NTB, 1, _TN), jnp.int32),
        scratch_shapes=[
            pltpu.VMEM((8, _TN), jnp.float32),
            pltpu.VMEM((8, _TN), jnp.int32),
        ],
    )(xt8, e2r)


def _gather_call(E2, codes_flat):
    info = plsc.get_sparse_core_info()
    nc, ns = info.num_cores, info.num_subcores
    nw = nc * ns
    tpw = _N // nw
    half = _K // 2
    mesh = plsc.VectorSubcoreMesh(core_axis_name="c", subcore_axis_name="s")

    @functools.partial(
        pl.kernel, mesh=mesh,
        compiler_params=pltpu.CompilerParams(use_tc_tiling_on_sc=False, needs_layout_passes=False),
        out_type=jax.ShapeDtypeStruct((nw * _C, tpw), jnp.float32),
        scratch_types=[
            pltpu.VMEM((tpw,), jnp.int32),
            pltpu.VMEM((half * _C // 128, 128), jnp.float32),
            pltpu.VMEM((_C, tpw), jnp.float32),
        ],
    )
    def gather_k(e2_hbm, codes_hbm, out_hbm, codes_v, et, outb):
        wid = lax.axis_index("s") * nc + lax.axis_index("c")
        pltpu.sync_copy(codes_hbm.at[pl.ds(wid * tpw, tpw)], codes_v)
        for p in range(2):
            pltpu.sync_copy(e2_hbm.at[pl.ds(p * (half * _C // 128),
                                            half * _C // 128)], et)

            def body(i, _):
                code = codes_v[pl.ds(i * 16, 16)]
                local = code - p * half
                lc = jnp.minimum(jnp.maximum(local, 0), half - 1)
                valid = (local >= 0) & (local < half)
                row = lc >> 5
                colbase = (lc & 31) * 4
                for c in range(4):
                    g = plsc.load_gather(et, [row, colbase + c])
                    if p == 0:
                        val = jnp.where(valid, g, 0.0)
                    else:
                        val = jnp.where(valid, g, outb[c, pl.ds(i * 16, 16)])
                    outb[c, pl.ds(i * 16, 16)] = val
                return 0

            lax.fori_loop(0, tpw // 16, body, 0)
        pltpu.sync_copy(outb, out_hbm.at[pl.ds(wid * _C, _C)])

    out = gather_k(E2, codes_flat)
    return out.reshape(nw, _C, tpw).transpose(0, 2, 1).reshape(_N, _C)


def _finish_body(xf_ref, zq_ref, zst_ref, q_ref, acc_ref):
    i = pl.program_id(0)
    xb = xf_ref[...]
    diff = zq_ref[...] - xb
    zst_ref[...] = xb + diff
    part = jnp.sum(diff * diff)

    @pl.when(i == 0)
    def _():
        acc_ref[0, 0] = part

    @pl.when(i > 0)
    def _():
        acc_ref[0, 0] += part

    @pl.when(i == _NFB - 1)
    def _():
        m = acc_ref[0, 0] * (1.0 / float(_N * _C))
        q_ref[0, 0] = m + _BETA * m


def _finish_call(xf, zq):
    return pl.pallas_call(
        _finish_body,
        grid=(_NFB,),
        in_specs=[
            pl.BlockSpec((_FB, _C), lambda i: (i, 0)),
            pl.BlockSpec((_FB, _C), lambda i: (i, 0)),
        ],
        out_specs=[
            pl.BlockSpec((_FB, _C), lambda i: (i, 0)),
            pl.BlockSpec(memory_space=pltpu.SMEM),
        ],
        out_shape=[
            jax.ShapeDtypeStruct((_N, _C), jnp.float32),
            jax.ShapeDtypeStruct((1, 1), jnp.float32),
        ],
        scratch_shapes=[pltpu.SMEM((1, 1), jnp.float32)],
    )(xf, zq)


def kernel(x, E):
    b, t, c, h, w = x.shape
    xf = jnp.transpose(x, (0, 1, 3, 4, 2)).reshape(-1, c)
    xt8 = jnp.concatenate(
        [xf.T, jnp.zeros((8 - _C, _N), jnp.float32)], axis=0)
    e2r = jnp.concatenate(
        [(2.0 * E).reshape(_NKCH, _TK, _C),
         jnp.zeros((_NKCH, _TK, 8 - _C), jnp.float32)], axis=2)

    codes3 = _codes_call(xt8, e2r)
    codes_flat = codes3.reshape(_N)

    zq_flat = _gather_call(E.reshape(_K * _C // 128, 128), codes_flat)

    zst_flat, q = _finish_call(xf, zq_flat)

    z_q_st = zst_flat.reshape(b, t, h, w, c).transpose(0, 1, 4, 2, 3)
    qloss = jnp.reshape(q, ())
    codes = codes_flat.reshape(b, t, h, w)
    return z_q_st, qloss, codes


# speedup vs baseline: 6.5464x; 1.0757x over previous
import functools

import jax
import jax.numpy as jnp
from jax import lax
from jax.experimental import pallas as pl
from jax.experimental.pallas import tpu as pltpu
from jax.experimental.pallas import tpu_sc as plsc

_K = 32768
_C = 4
_N = 65536
_TN = 1024
_TK = 1024
_NKCH = _K // _TK
_NTB = _N // _TN
_BIG = 2**30
_BETA = 0.25

_NFB = 8
_FB = _N // _NFB

_GR = 128
_NGR = _N // _GR


def _argmin_body(xt_ref, ek_ref, codes_ref, bd_ref, bi_ref):
    k = pl.program_id(1)
    xb16 = xt_ref[...]
    ech = ek_ref[0]
    s = lax.dot_general(
        ech, xb16, (((1,), (0,)), ((), ())),
        preferred_element_type=jnp.float32)
    accv = jnp.full((8, _TN), -jnp.inf, jnp.float32)
    acci = jnp.zeros((8, _TN), jnp.int32)
    for v in range(_TK // 8):
        sv = s[v * 8:(v + 1) * 8, :]
        mask = sv > accv
        accv = jnp.where(mask, sv, accv)
        acci = jnp.where(mask, v, acci)
    siota = lax.broadcasted_iota(jnp.int32, (8, _TN), 0)
    bidxc = k * _TK + (acci * 8 + siota)
    prev_d = jnp.where(k == 0, -jnp.inf, bd_ref[...])
    prev_i = jnp.where(k == 0, 0, bi_ref[...])
    better = accv > prev_d
    bd_ref[...] = jnp.where(better, accv, prev_d)
    bi_ref[...] = jnp.where(better, bidxc, prev_i)

    @pl.when(k == _NKCH - 1)
    def _():
        av = bd_ref[...]
        ai = bi_ref[...]
        bfin = jnp.max(av, axis=0)
        codes_ref[0, 0, :] = jnp.min(
            jnp.where(av == bfin[None, :], ai, 2**30), axis=0)


def _codes_call(xt8, e2r):
    return pl.pallas_call(
        _argmin_body,
        grid=(_NTB, _NKCH),
        in_specs=[
            pl.BlockSpec((8, _TN), lambda i, k: (0, i)),
            pl.BlockSpec((1, _TK, 8), lambda i, k: (k, 0, 0)),
        ],
        out_specs=pl.BlockSpec((1, 1, _TN), lambda i, k: (i, 0, 0)),
        out_shape=jax.ShapeDtypeStruct((_NTB, 1, _TN), jnp.int32),
        scratch_shapes=[
            pltpu.VMEM((8, _TN), jnp.float32),
            pltpu.VMEM((8, _TN), jnp.int32),
        ],
    )(xt8, e2r)


def _gather_call(E2, codes_flat):
    info = plsc.get_sparse_core_info()
    nc, ns = info.num_cores, info.num_subcores
    nw = nc * ns
    tpw = _N // nw
    half = _K // 2
    mesh = plsc.VectorSubcoreMesh(core_axis_name="c", subcore_axis_name="s")

    @functools.partial(
        pl.kernel, mesh=mesh,
        compiler_params=pltpu.CompilerParams(use_tc_tiling_on_sc=False, needs_layout_passes=False),
        out_type=jax.ShapeDtypeStruct((nw * _C, tpw), jnp.float32),
        scratch_types=[
            pltpu.VMEM((tpw,), jnp.int32),
            pltpu.VMEM((half * _C // 128, 128), jnp.float32),
            pltpu.VMEM((_C, tpw), jnp.float32),
        ],
    )
    def gather_k(e2_hbm, codes_hbm, out_hbm, codes_v, et, outb):
        wid = lax.axis_index("s") * nc + lax.axis_index("c")
        pltpu.sync_copy(codes_hbm.at[pl.ds(wid * tpw, tpw)], codes_v)
        for p in range(2):
            pltpu.sync_copy(e2_hbm.at[pl.ds(p * (half * _C // 128),
                                            half * _C // 128)], et)

            def body(i, _):
                code = codes_v[pl.ds(i * 16, 16)]
                local = code - p * half
                lc = jnp.minimum(jnp.maximum(local, 0), half - 1)
                valid = (local >= 0) & (local < half)
                row = lc >> 5
                colbase = (lc & 31) * 4
                for c in range(4):
                    g = plsc.load_gather(et, [row, colbase + c])
                    if p == 0:
                        val = jnp.where(valid, g, 0.0)
                    else:
                        val = jnp.where(valid, g, outb[c, pl.ds(i * 16, 16)])
                    outb[c, pl.ds(i * 16, 16)] = val
                return 0

            lax.fori_loop(0, tpw // 16, body, 0)
        pltpu.sync_copy(outb, out_hbm.at[pl.ds(wid * _C, _C)])

    out = gather_k(E2, codes_flat)
    return out.reshape(nw, _C, tpw).transpose(0, 2, 1).reshape(_N, _C)


def _finish_body(xf_ref, zq_ref, zst_ref, q_ref, acc_ref):
    i = pl.program_id(0)
    xb = xf_ref[...]
    diff = zq_ref[...] - xb
    zst_ref[...] = xb + diff
    part = jnp.sum(diff * diff)

    @pl.when(i == 0)
    def _():
        acc_ref[0, 0] = part

    @pl.when(i > 0)
    def _():
        acc_ref[0, 0] += part

    @pl.when(i == _NFB - 1)
    def _():
        m = acc_ref[0, 0] * (1.0 / float(_N * _C))
        q_ref[0, 0] = m + _BETA * m


def _finish_call(xf, zq):
    return pl.pallas_call(
        _finish_body,
        grid=(_NFB,),
        in_specs=[
            pl.BlockSpec((_FB, _C), lambda i: (i, 0)),
            pl.BlockSpec((_FB, _C), lambda i: (i, 0)),
        ],
        out_specs=[
            pl.BlockSpec((_FB, _C), lambda i: (i, 0)),
            pl.BlockSpec(memory_space=pltpu.SMEM),
        ],
        out_shape=[
            jax.ShapeDtypeStruct((_N, _C), jnp.float32),
            jax.ShapeDtypeStruct((1, 1), jnp.float32),
        ],
        scratch_shapes=[pltpu.SMEM((1, 1), jnp.float32)],
    )(xf, zq)


def kernel(x, E):
    b, t, c, h, w = x.shape
    xf = jnp.transpose(x, (0, 1, 3, 4, 2)).reshape(-1, c)
    xt8 = jnp.concatenate(
        [xf.T, -jnp.ones((1, _N), jnp.float32),
         jnp.zeros((3, _N), jnp.float32)], axis=0).astype(jnp.bfloat16)
    esq = jnp.sum(E * E, axis=1).reshape(_NKCH, _TK, 1)
    e2r = jnp.concatenate(
        [(2.0 * E).reshape(_NKCH, _TK, _C), esq,
         jnp.zeros((_NKCH, _TK, 3), jnp.float32)], axis=2)

    codes3 = _codes_call(xt8, e2r)
    codes_flat = codes3.reshape(_N)

    zq_flat = _gather_call(E.reshape(_K * _C // 128, 128), codes_flat)

    zst_flat, q = _finish_call(xf, zq_flat)

    z_q_st = zst_flat.reshape(b, t, h, w, c).transpose(0, 1, 4, 2, 3)
    qloss = jnp.reshape(q, ())
    codes = codes_flat.reshape(b, t, h, w)
    return z_q_st, qloss, codes


# speedup vs baseline: 8.9066x; 1.3605x over previous
import functools

import jax
import jax.numpy as jnp
from jax import lax
from jax.experimental import pallas as pl
from jax.experimental.pallas import tpu as pltpu
from jax.experimental.pallas import tpu_sc as plsc

_K = 32768
_C = 4
_N = 65536
_TN = 2048
_TK = 1024
_NKCH = _K // _TK
_NTB = _N // _TN
_BIG = 2**30
_BETA = 0.25

_NFB = 8
_FB = _N // _NFB

_GR = 128
_NGR = _N // _GR


def _argmin_body(xt_ref, ek_ref, codes_ref, bd_ref, bi_ref):
    k = pl.program_id(1)
    xb16 = xt_ref[...]
    ech = ek_ref[0]
    s = lax.dot_general(
        ech, xb16, (((1,), (0,)), ((), ())),
        preferred_element_type=jnp.float32)
    accv = jnp.full((8, _TN), -jnp.inf, jnp.float32)
    acci = jnp.zeros((8, _TN), jnp.int32)
    for v in range(_TK // 8):
        sv = s[v * 8:(v + 1) * 8, :]
        mask = sv > accv
        accv = jnp.where(mask, sv, accv)
        acci = jnp.where(mask, v, acci)
    siota = lax.broadcasted_iota(jnp.int32, (8, _TN), 0)
    bidxc = k * _TK + (acci * 8 + siota)
    prev_d = jnp.where(k == 0, -jnp.inf, bd_ref[...])
    prev_i = jnp.where(k == 0, 0, bi_ref[...])
    better = accv > prev_d
    bd_ref[...] = jnp.where(better, accv, prev_d)
    bi_ref[...] = jnp.where(better, bidxc, prev_i)

    @pl.when(k == _NKCH - 1)
    def _():
        av = bd_ref[...]
        ai = bi_ref[...]
        bfin = jnp.max(av, axis=0)
        codes_ref[0, 0, :] = jnp.min(
            jnp.where(av == bfin[None, :], ai, 2**30), axis=0)


def _codes_call(xt8, e2r):
    return pl.pallas_call(
        _argmin_body,
        grid=(_NTB, _NKCH),
        in_specs=[
            pl.BlockSpec((8, _TN), lambda i, k: (0, i)),
            pl.BlockSpec((1, _TK, 8), lambda i, k: (k, 0, 0)),
        ],
        out_specs=pl.BlockSpec((1, 1, _TN), lambda i, k: (i, 0, 0)),
        out_shape=jax.ShapeDtypeStruct((_NTB, 1, _TN), jnp.int32),
        scratch_shapes=[
            pltpu.VMEM((8, _TN), jnp.float32),
            pltpu.VMEM((8, _TN), jnp.int32),
        ],
    )(xt8, e2r)


def _gather_call(E2, codes_flat):
    info = plsc.get_sparse_core_info()
    nc, ns = info.num_cores, info.num_subcores
    nw = nc * ns
    tpw = _N // nw
    half = _K // 2
    mesh = plsc.VectorSubcoreMesh(core_axis_name="c", subcore_axis_name="s")

    @functools.partial(
        pl.kernel, mesh=mesh,
        compiler_params=pltpu.CompilerParams(use_tc_tiling_on_sc=False, needs_layout_passes=False),
        out_type=jax.ShapeDtypeStruct((nw * _C, tpw), jnp.float32),
        scratch_types=[
            pltpu.VMEM((tpw,), jnp.int32),
            pltpu.VMEM((half * _C // 128, 128), jnp.float32),
            pltpu.VMEM((_C, tpw), jnp.float32),
        ],
    )
    def gather_k(e2_hbm, codes_hbm, out_hbm, codes_v, et, outb):
        wid = lax.axis_index("s") * nc + lax.axis_index("c")
        pltpu.sync_copy(codes_hbm.at[pl.ds(wid * tpw, tpw)], codes_v)
        for p in range(2):
            pltpu.sync_copy(e2_hbm.at[pl.ds(p * (half * _C // 128),
                                            half * _C // 128)], et)

            def body(i, _):
                code = codes_v[pl.ds(i * 16, 16)]
                local = code - p * half
                lc = jnp.minimum(jnp.maximum(local, 0), half - 1)
                valid = (local >= 0) & (local < half)
                row = lc >> 5
                colbase = (lc & 31) * 4
                for c in range(4):
                    g = plsc.load_gather(et, [row, colbase + c])
                    if p == 0:
                        val = jnp.where(valid, g, 0.0)
                    else:
                        val = jnp.where(valid, g, outb[c, pl.ds(i * 16, 16)])
                    outb[c, pl.ds(i * 16, 16)] = val
                return 0

            lax.fori_loop(0, tpw // 16, body, 0)
        pltpu.sync_copy(outb, out_hbm.at[pl.ds(wid * _C, _C)])

    out = gather_k(E2, codes_flat)
    return out.reshape(nw, _C, tpw).transpose(0, 2, 1).reshape(_N, _C)


def _finish_body(xf_ref, zq_ref, zst_ref, q_ref, acc_ref):
    i = pl.program_id(0)
    xb = xf_ref[...]
    diff = zq_ref[...] - xb
    zst_ref[...] = xb + diff
    part = jnp.sum(diff * diff)

    @pl.when(i == 0)
    def _():
        acc_ref[0, 0] = part

    @pl.when(i > 0)
    def _():
        acc_ref[0, 0] += part

    @pl.when(i == _NFB - 1)
    def _():
        m = acc_ref[0, 0] * (1.0 / float(_N * _C))
        q_ref[0, 0] = m + _BETA * m


def _finish_call(xf, zq):
    return pl.pallas_call(
        _finish_body,
        grid=(_NFB,),
        in_specs=[
            pl.BlockSpec((_FB, _C), lambda i: (i, 0)),
            pl.BlockSpec((_FB, _C), lambda i: (i, 0)),
        ],
        out_specs=[
            pl.BlockSpec((_FB, _C), lambda i: (i, 0)),
            pl.BlockSpec(memory_space=pltpu.SMEM),
        ],
        out_shape=[
            jax.ShapeDtypeStruct((_N, _C), jnp.float32),
            jax.ShapeDtypeStruct((1, 1), jnp.float32),
        ],
        scratch_shapes=[pltpu.SMEM((1, 1), jnp.float32)],
    )(xf, zq)


def kernel(x, E):
    b, t, c, h, w = x.shape
    xf = jnp.transpose(x, (0, 1, 3, 4, 2)).reshape(-1, c)
    xt8 = jnp.concatenate(
        [xf.T, -jnp.ones((1, _N), jnp.float32),
         jnp.zeros((3, _N), jnp.float32)], axis=0).astype(jnp.bfloat16)
    esq = jnp.sum(E * E, axis=1).reshape(_NKCH, _TK, 1)
    e2r = jnp.concatenate(
        [(2.0 * E).reshape(_NKCH, _TK, _C), esq,
         jnp.zeros((_NKCH, _TK, 3), jnp.float32)], axis=2)

    codes3 = _codes_call(xt8, e2r)
    codes_flat = codes3.reshape(_N)

    zq_flat = _gather_call(E.reshape(_K * _C // 128, 128), codes_flat)

    zst_flat, q = _finish_call(xf, zq_flat)

    z_q_st = zst_flat.reshape(b, t, h, w, c).transpose(0, 1, 4, 2, 3)
    qloss = jnp.reshape(q, ())
    codes = codes_flat.reshape(b, t, h, w)
    return z_q_st, qloss, codes


# speedup vs baseline: 9.4958x; 1.0662x over previous
import functools

import jax
import jax.numpy as jnp
from jax import lax
from jax.experimental import pallas as pl
from jax.experimental.pallas import tpu as pltpu
from jax.experimental.pallas import tpu_sc as plsc

_K = 32768
_C = 4
_N = 65536
_TN = 4096
_TK = 1024
_NKCH = _K // _TK
_NTB = _N // _TN
_BIG = 2**30
_BETA = 0.25

_NFB = 8
_FB = _N // _NFB

_GR = 128
_NGR = _N // _GR


def _argmin_body(xt_ref, ek_ref, codes_ref, bd_ref, bi_ref):
    k = pl.program_id(1)
    xb16 = xt_ref[...]
    ech = ek_ref[0]
    s = lax.dot_general(
        ech, xb16, (((1,), (0,)), ((), ())),
        preferred_element_type=jnp.float32)
    accv = jnp.full((8, _TN), -jnp.inf, jnp.float32)
    acci = jnp.zeros((8, _TN), jnp.int32)
    for v in range(_TK // 8):
        sv = s[v * 8:(v + 1) * 8, :]
        mask = sv > accv
        accv = jnp.where(mask, sv, accv)
        acci = jnp.where(mask, v, acci)
    siota = lax.broadcasted_iota(jnp.int32, (8, _TN), 0)
    bidxc = k * _TK + (acci * 8 + siota)
    prev_d = jnp.where(k == 0, -jnp.inf, bd_ref[...])
    prev_i = jnp.where(k == 0, 0, bi_ref[...])
    better = accv > prev_d
    bd_ref[...] = jnp.where(better, accv, prev_d)
    bi_ref[...] = jnp.where(better, bidxc, prev_i)

    @pl.when(k == _NKCH - 1)
    def _():
        av = bd_ref[...]
        ai = bi_ref[...]
        bfin = jnp.max(av, axis=0)
        codes_ref[0, 0, :] = jnp.min(
            jnp.where(av == bfin[None, :], ai, 2**30), axis=0)


def _codes_call(xt8, e2r):
    return pl.pallas_call(
        _argmin_body,
        grid=(_NTB, _NKCH),
        in_specs=[
            pl.BlockSpec((8, _TN), lambda i, k: (0, i)),
            pl.BlockSpec((1, _TK, 8), lambda i, k: (k, 0, 0)),
        ],
        out_specs=pl.BlockSpec((1, 1, _TN), lambda i, k: (i, 0, 0)),
        out_shape=jax.ShapeDtypeStruct((_NTB, 1, _TN), jnp.int32),
        scratch_shapes=[
            pltpu.VMEM((8, _TN), jnp.float32),
            pltpu.VMEM((8, _TN), jnp.int32),
        ],
    )(xt8, e2r)


def _gather_call(E2, codes_flat):
    info = plsc.get_sparse_core_info()
    nc, ns = info.num_cores, info.num_subcores
    nw = nc * ns
    tpw = _N // nw
    half = _K // 2
    mesh = plsc.VectorSubcoreMesh(core_axis_name="c", subcore_axis_name="s")

    @functools.partial(
        pl.kernel, mesh=mesh,
        compiler_params=pltpu.CompilerParams(use_tc_tiling_on_sc=False, needs_layout_passes=False),
        out_type=jax.ShapeDtypeStruct((nw * _C, tpw), jnp.float32),
        scratch_types=[
            pltpu.VMEM((tpw,), jnp.int32),
            pltpu.VMEM((half * _C // 128, 128), jnp.float32),
            pltpu.VMEM((_C, tpw), jnp.float32),
        ],
    )
    def gather_k(e2_hbm, codes_hbm, out_hbm, codes_v, et, outb):
        wid = lax.axis_index("s") * nc + lax.axis_index("c")
        pltpu.sync_copy(codes_hbm.at[pl.ds(wid * tpw, tpw)], codes_v)
        for p in range(2):
            pltpu.sync_copy(e2_hbm.at[pl.ds(p * (half * _C // 128),
                                            half * _C // 128)], et)

            def body(i, _):
                code = codes_v[pl.ds(i * 16, 16)]
                local = code - p * half
                lc = jnp.minimum(jnp.maximum(local, 0), half - 1)
                valid = (local >= 0) & (local < half)
                row = lc >> 5
                colbase = (lc & 31) * 4
                for c in range(4):
                    g = plsc.load_gather(et, [row, colbase + c])
                    if p == 0:
                        val = jnp.where(valid, g, 0.0)
                    else:
                        val = jnp.where(valid, g, outb[c, pl.ds(i * 16, 16)])
                    outb[c, pl.ds(i * 16, 16)] = val
                return 0

            lax.fori_loop(0, tpw // 16, body, 0)
        pltpu.sync_copy(outb, out_hbm.at[pl.ds(wid * _C, _C)])

    out = gather_k(E2, codes_flat)
    return out.reshape(nw, _C, tpw).transpose(0, 2, 1).reshape(_N, _C)


def _finish_body(xf_ref, zq_ref, zst_ref, q_ref, acc_ref):
    i = pl.program_id(0)
    xb = xf_ref[...]
    diff = zq_ref[...] - xb
    zst_ref[...] = xb + diff
    part = jnp.sum(diff * diff)

    @pl.when(i == 0)
    def _():
        acc_ref[0, 0] = part

    @pl.when(i > 0)
    def _():
        acc_ref[0, 0] += part

    @pl.when(i == _NFB - 1)
    def _():
        m = acc_ref[0, 0] * (1.0 / float(_N * _C))
        q_ref[0, 0] = m + _BETA * m


def _finish_call(xf, zq):
    return pl.pallas_call(
        _finish_body,
        grid=(_NFB,),
        in_specs=[
            pl.BlockSpec((_FB, _C), lambda i: (i, 0)),
            pl.BlockSpec((_FB, _C), lambda i: (i, 0)),
        ],
        out_specs=[
            pl.BlockSpec((_FB, _C), lambda i: (i, 0)),
            pl.BlockSpec(memory_space=pltpu.SMEM),
        ],
        out_shape=[
            jax.ShapeDtypeStruct((_N, _C), jnp.float32),
            jax.ShapeDtypeStruct((1, 1), jnp.float32),
        ],
        scratch_shapes=[pltpu.SMEM((1, 1), jnp.float32)],
    )(xf, zq)


def kernel(x, E):
    b, t, c, h, w = x.shape
    xf = jnp.transpose(x, (0, 1, 3, 4, 2)).reshape(-1, c)
    xt8 = jnp.concatenate(
        [xf.T, -jnp.ones((1, _N), jnp.float32),
         jnp.zeros((3, _N), jnp.float32)], axis=0).astype(jnp.bfloat16)
    esq = jnp.sum(E * E, axis=1).reshape(_NKCH, _TK, 1)
    e2r = jnp.concatenate(
        [(2.0 * E).reshape(_NKCH, _TK, _C), esq,
         jnp.zeros((_NKCH, _TK, 3), jnp.float32)], axis=2)

    codes3 = _codes_call(xt8, e2r)
    codes_flat = codes3.reshape(_N)

    zq_flat = _gather_call(E.reshape(_K * _C // 128, 128), codes_flat)

    zst_flat, q = _finish_call(xf, zq_flat)

    z_q_st = zst_flat.reshape(b, t, h, w, c).transpose(0, 1, 4, 2, 3)
    qloss = jnp.reshape(q, ())
    codes = codes_flat.reshape(b, t, h, w)
    return z_q_st, qloss, codes
